# baseline (speedup 1.0000x reference)
import functools

import jax
import jax.numpy as jnp
from jax import lax
from jax.experimental import pallas as pl
from jax.experimental.pallas import tpu as pltpu
from jax.experimental.pallas import tpu_sc as plsc

VOCAB = 1000000
EMBED_DIM = 64
NUM_PROMPT = 3
PID_BASE = VOCAB - NUM_PROMPT

NC, NS, L = 2, 16, 16
NW = NC * NS
CHUNK = 1024
GATHER_W = 128
GPC = CHUNK // GATHER_W


def _sc_body(idx_hbm, table_hbm, pp_hbm, out_hbm, idx_v, rows_v, pp_v, sem):
    n_rows = idx_hbm.shape[0] * idx_hbm.shape[1]
    rows_per_w = n_rows // NW
    chunks_per_w = rows_per_w // CHUNK

    wid = lax.axis_index("s") * NC + lax.axis_index("c")
    base_row = wid * rows_per_w

    pltpu.sync_copy(pp_hbm, pp_v)

    iota16 = lax.iota(jnp.int32, L)

    def do_chunk(g, _):
        row_off = pl.multiple_of(base_row + g * CHUNK, CHUNK)
        pltpu.sync_copy(
            idx_hbm.at[pl.ds(pl.multiple_of(row_off // GATHER_W, 8), GPC)],
            idx_v)
        for j in range(GPC):
            pltpu.async_copy(
                table_hbm.at[idx_v.at[j]],
                rows_v.at[pl.ds(j * GATHER_W, GATHER_W)],
                sem,
            )
        for j in range(GPC):
            pltpu.make_async_copy(
                table_hbm.at[idx_v.at[j]],
                rows_v.at[pl.ds(j * GATHER_W, GATHER_W)],
                sem,
            ).wait()

        def fix_group(g2, _):
            jj = g2 // (GATHER_W // L)
            ll = g2 % (GATHER_W // L)
            ivec = idx_v[jj, pl.ds(ll * L, L)]
            cond = ivec >= PID_BASE
            cnt = jnp.sum(cond.astype(jnp.int32))

            @pl.when(cnt > 0)
            def _():
                def fix_lane(_, mask):
                    lane = jnp.min(jnp.where(mask > 0, iota16, L))
                    k = jnp.max(jnp.where(iota16 == lane, ivec - PID_BASE, -1))
                    row = g2 * L + lane
                    row_splat = jnp.broadcast_to(row, (L,)).astype(jnp.int32)
                    for q in range(EMBED_DIM // L):
                        val = pp_v[pl.ds(k * EMBED_DIM + q * L, L)]
                        plsc.store_scatter(
                            rows_v, [row_splat, iota16 + q * L], val)
                    return mask & (iota16 != lane).astype(jnp.int32)

                lax.fori_loop(0, cnt, fix_lane, cond.astype(jnp.int32))

            return 0

        lax.fori_loop(0, CHUNK // L, fix_group, 0)

        pltpu.sync_copy(rows_v, out_hbm.at[pl.ds(row_off, CHUNK)])
        return 0

    lax.fori_loop(0, chunks_per_w, do_chunk, 0)


@jax.jit
def _run(idx2d, table, pp_flat):
    n_rows = idx2d.shape[0] * idx2d.shape[1]
    mesh = plsc.VectorSubcoreMesh(core_axis_name="c", subcore_axis_name="s")
    return pl.kernel(
        _sc_body,
        out_type=jax.ShapeDtypeStruct((n_rows, EMBED_DIM), jnp.float32),
        mesh=mesh,
        scratch_types=[
            pltpu.VMEM((GPC, GATHER_W), jnp.int32),
            pltpu.VMEM((CHUNK, EMBED_DIM), jnp.float32),
            pltpu.VMEM((NUM_PROMPT * EMBED_DIM,), jnp.float32),
            pltpu.SemaphoreType.DMA,
        ],
        compiler_params=pltpu.CompilerParams(
            use_tc_tiling_on_sc=False, needs_layout_passes=False),
    )(idx2d, table, pp_flat)


def kernel(input, table, prompt_params):
    b, l = input.shape
    idx2d = input.astype(jnp.int32).reshape(-1, GATHER_W)
    out = _run(idx2d, table, prompt_params.reshape(-1))
    return out.reshape(b, l, EMBED_DIM)
